# baseline (speedup 1.0000x reference)
import jax
import jax.numpy as jnp
from jax.experimental import pallas as pl

N_TOKENS = 8192
DIM = 32
N_CODES = 8192
BLOCK_N = 256
CHUNK = 2048


def _vq_kernel(x_ref, cb_ref, x2_ref, c2_ref, q_ref, idx_ref):
    x = x_ref[...]
    cb = cb_ref[...]
    x2 = x2_ref[...]
    c2 = c2_ref[...]

    mm = jax.lax.dot_general(
        x.astype(jnp.bfloat16), cb.astype(jnp.bfloat16), (((1,), (1,)), ((), ())),
        preferred_element_type=jnp.float32,
    )
    d2 = x2 + c2 - 2.0 * mm
    dist = jnp.sqrt(jnp.maximum(d2, 0.0))

    acc_v = jnp.full((BLOCK_N, 1), jnp.inf, dtype=jnp.float32)
    acc_i = jnp.zeros((BLOCK_N, 1), dtype=jnp.int32)
    for c in range(N_CODES // CHUNK):
        sub = jax.lax.slice_in_dim(dist, c * CHUNK, (c + 1) * CHUNK, axis=1)
        cv = jnp.min(sub, axis=1, keepdims=True)
        iota = jax.lax.broadcasted_iota(jnp.int32, sub.shape, 1) + c * CHUNK
        ci = jnp.min(jnp.where(sub == cv, iota, N_CODES), axis=1, keepdims=True)
        take = cv < acc_v
        acc_v = jnp.where(take, cv, acc_v).astype(jnp.bfloat16).astype(jnp.float32)
        acc_i = jnp.where(take, ci, acc_i)

    idx_ref[...] = acc_i[:, 0]
    q_ref[...] = x


def kernel(inputs, codebook, residuals_param):
    del residuals_param
    x2 = jnp.sum(inputs * inputs, axis=1, keepdims=True)
    c2 = jnp.sum(codebook * codebook, axis=1)[None, :]

    grid = (N_TOKENS // BLOCK_N,)
    quantized, indices = pl.pallas_call(
        _vq_kernel,
        grid=grid,
        in_specs=[
            pl.BlockSpec((BLOCK_N, DIM), lambda i: (i, 0)),
            pl.BlockSpec((N_CODES, DIM), lambda i: (0, 0)),
            pl.BlockSpec((BLOCK_N, 1), lambda i: (i, 0)),
            pl.BlockSpec((1, N_CODES), lambda i: (0, 0)),
        ],
        out_specs=[
            pl.BlockSpec((BLOCK_N, DIM), lambda i: (i, 0)),
            pl.BlockSpec((BLOCK_N,), lambda i: (i,)),
        ],
        out_shape=[
            jax.ShapeDtypeStruct((N_TOKENS, DIM), jnp.float32),
            jax.ShapeDtypeStruct((N_TOKENS,), jnp.int32),
        ],
    )(inputs, codebook, x2, c2)
    return (quantized, indices)


# speedup vs baseline: 1.1383x; 1.1383x over previous
import jax
import jax.numpy as jnp
from jax.experimental import pallas as pl

N_TOKENS = 8192
DIM = 32
N_CODES = 8192
BLOCK_N = 256
CHUNK = 2048
LANES = 128
BIG = 2**30


def _vq_kernel(x_ref, cb2_ref, x2_ref, c2_ref, q_ref, idx_ref):
    x = x_ref[...]
    cb2 = cb2_ref[...]
    x2 = x2_ref[...]
    c2 = c2_ref[...]

    mm2 = jax.lax.dot_general(
        x.astype(jnp.bfloat16), cb2, (((1,), (1,)), ((), ())),
        preferred_element_type=jnp.float32,
    )

    lane = jax.lax.broadcasted_iota(jnp.int32, (BLOCK_N, LANES), 1)
    acc_v = jnp.full((BLOCK_N, 1), jnp.inf, dtype=jnp.float32)
    acc_i = jnp.zeros((BLOCK_N, 1), dtype=jnp.int32)
    for c in range(N_CODES // CHUNK):
        lane_v = jnp.full((BLOCK_N, LANES), jnp.inf, dtype=jnp.float32)
        lane_t = jnp.zeros((BLOCK_N, LANES), dtype=jnp.int32)
        for t in range(CHUNK // LANES):
            col = c * CHUNK + t * LANES
            mmt = jax.lax.slice_in_dim(mm2, col, col + LANES, axis=1)
            c2t = jax.lax.slice_in_dim(c2, col, col + LANES, axis=1)
            d2 = (x2 + c2t) - mmt
            dt = jnp.sqrt(jnp.maximum(d2, 0.0))
            m = dt < lane_v
            lane_v = jnp.where(m, dt, lane_v)
            lane_t = jnp.where(m, jnp.int32(t), lane_t)
        cv = jnp.min(lane_v, axis=1, keepdims=True)
        cand = jnp.where(lane_v == cv, lane_t * LANES + lane, BIG)
        ci = jnp.min(cand, axis=1, keepdims=True) + c * CHUNK
        take = cv < acc_v
        acc_v = jnp.where(take, cv, acc_v).astype(jnp.bfloat16).astype(jnp.float32)
        acc_i = jnp.where(take, ci, acc_i)

    idx_ref[...] = acc_i[:, 0]
    q_ref[...] = x


def kernel(inputs, codebook, residuals_param):
    del residuals_param
    x2 = jnp.sum(inputs * inputs, axis=1, keepdims=True)
    c2 = jnp.sum(codebook * codebook, axis=1)[None, :]
    cb2 = codebook.astype(jnp.bfloat16) * 2

    grid = (N_TOKENS // BLOCK_N,)
    quantized, indices = pl.pallas_call(
        _vq_kernel,
        grid=grid,
        in_specs=[
            pl.BlockSpec((BLOCK_N, DIM), lambda i: (i, 0)),
            pl.BlockSpec((N_CODES, DIM), lambda i: (0, 0)),
            pl.BlockSpec((BLOCK_N, 1), lambda i: (i, 0)),
            pl.BlockSpec((1, N_CODES), lambda i: (0, 0)),
        ],
        out_specs=[
            pl.BlockSpec((BLOCK_N, DIM), lambda i: (i, 0)),
            pl.BlockSpec((BLOCK_N,), lambda i: (i,)),
        ],
        out_shape=[
            jax.ShapeDtypeStruct((N_TOKENS, DIM), jnp.float32),
            jax.ShapeDtypeStruct((N_TOKENS,), jnp.int32),
        ],
    )(inputs, cb2, x2, c2)
    return (quantized, indices)


# speedup vs baseline: 1.4824x; 1.3023x over previous
import jax
import jax.numpy as jnp
from jax.experimental import pallas as pl

N_TOKENS = 8192
DIM = 32
N_CODES = 8192
BLOCK_N = 256
CHUNK = 2048
LANES = 128
BIG = 2**30


def _vq_kernel(x_ref, cb2_ref, x2_ref, c2_ref, q_ref, idx_ref):
    x = x_ref[...]
    cb2 = cb2_ref[...]
    x2 = x2_ref[...]
    c2 = c2_ref[...]

    mm2 = jax.lax.dot_general(
        x.astype(jnp.bfloat16), cb2, (((1,), (1,)), ((), ())),
        preferred_element_type=jnp.float32,
    )

    lane = jax.lax.broadcasted_iota(jnp.int32, (BLOCK_N, LANES), 1)
    acc_v = jnp.full((BLOCK_N, 1), jnp.inf, dtype=jnp.float32)
    acc_i = jnp.zeros((BLOCK_N, 1), dtype=jnp.int32)
    for c in range(N_CODES // CHUNK):
        lane_v = None
        lane_t = None
        for t in range(CHUNK // LANES):
            col = c * CHUNK + t * LANES
            mmt = jax.lax.slice_in_dim(mm2, col, col + LANES, axis=1)
            c2t = jax.lax.slice_in_dim(c2, col, col + LANES, axis=1)
            d2 = (x2 + c2t) - mmt
            dt = jnp.where(d2 <= 0.0, 0.0, d2 * jax.lax.rsqrt(d2))
            if lane_v is None:
                lane_v = dt
                lane_t = jnp.zeros((BLOCK_N, LANES), dtype=jnp.int32)
            else:
                m = dt < lane_v
                lane_v = jnp.where(m, dt, lane_v)
                lane_t = jnp.where(m, jnp.int32(t), lane_t)
        cv = jnp.min(lane_v, axis=1, keepdims=True)
        cand = jnp.where(lane_v == cv, lane_t * LANES + lane, BIG)
        ci = jnp.min(cand, axis=1, keepdims=True) + c * CHUNK
        take = cv < acc_v
        acc_v = jnp.where(take, cv, acc_v).astype(jnp.bfloat16).astype(jnp.float32)
        acc_i = jnp.where(take, ci, acc_i)

    idx_ref[...] = acc_i[:, 0]
    q_ref[...] = x


def kernel(inputs, codebook, residuals_param):
    del residuals_param
    x2 = jnp.sum(inputs * inputs, axis=1, keepdims=True)
    c2 = jnp.sum(codebook * codebook, axis=1)[None, :]
    cb2 = codebook.astype(jnp.bfloat16) * 2

    grid = (N_TOKENS // BLOCK_N,)
    quantized, indices = pl.pallas_call(
        _vq_kernel,
        grid=grid,
        in_specs=[
            pl.BlockSpec((BLOCK_N, DIM), lambda i: (i, 0)),
            pl.BlockSpec((N_CODES, DIM), lambda i: (0, 0)),
            pl.BlockSpec((BLOCK_N, 1), lambda i: (i, 0)),
            pl.BlockSpec((1, N_CODES), lambda i: (0, 0)),
        ],
        out_specs=[
            pl.BlockSpec((BLOCK_N, DIM), lambda i: (i, 0)),
            pl.BlockSpec((BLOCK_N,), lambda i: (i,)),
        ],
        out_shape=[
            jax.ShapeDtypeStruct((N_TOKENS, DIM), jnp.float32),
            jax.ShapeDtypeStruct((N_TOKENS,), jnp.int32),
        ],
    )(inputs, cb2, x2, c2)
    return (quantized, indices)
